# Initial kernel scaffold; baseline (speedup 1.0000x reference)
import jax
import jax.numpy as jnp
from jax.experimental import pallas as pl


def kernel(batch_heads, batch_rels, batch_tails, batch_ids, fact_ids, weight_list, fact_val):
    raise NotImplementedError("write your pallas kernel here")


# speedup vs baseline: 6.9099x; 6.9099x over previous
import jax
import jax.numpy as jnp
from jax import lax
from jax.experimental import pallas as pl
from jax.experimental.pallas import tpu as pltpu, tpu_sc as plsc

_NE = 100_000
_NRB = 1_600
_NF = 3_200_000
_D = 16
_NREL = 200
_NC = 2
_NS = 16
_NW = _NC * _NS
_CHUNK = 1024
_SUB = 128
_NSUB = _CHUNK // _SUB
_NCHUNKS = _NF // _CHUNK
_IDXROWS = _NF // _SUB
_EROWS_PER_TILE = _NE // _NS
_RROWS_PER_TILE = _NRB // _NS
_OUT_ROWS = 2 * _NE + _NRB + _NF
_FACT_OFF = 2 * _NE + _NRB
_HROWS_PER_W = _NE // _NW
_RROWS_PER_W = _NRB // _NW


def _zero_rows(buf, n):
    z = jnp.zeros((_D,), jnp.float32)

    def body(r, carry):
        buf[r] = z
        return carry

    lax.fori_loop(0, n, body, 0)


_ECHUNK = 1000
_NECHUNK = _NE // _ECHUNK
_RCHUNK = 200
_NRCHUNK = _NRB // _RCHUNK


def _p1_body(heads2, tails2, rels1, ids1, w1, val2,
             head_out, tail_out, rel_out,
             acc_sh, rel_sh, val_v, w_v, sidx_v, rels_v, ids_v,
             ridx_v, sem):
    c = lax.axis_index("c")
    s = lax.axis_index("s")

    _zero_rows(val_v, _CHUNK)
    nz = (_NECHUNK // _NS) + jnp.where(s < _NECHUNK % _NS, 1, 0)

    def zbody(i, carry):
        pltpu.sync_copy(val_v.at[pl.ds(0, _ECHUNK)],
                        acc_sh.at[pl.ds((s + i * _NS) * _ECHUNK, _ECHUNK)])
        return carry

    lax.fori_loop(0, nz, zbody, 0)

    @pl.when(s < _NRCHUNK)
    def _():
        pltpu.sync_copy(val_v.at[pl.ds(0, _RCHUNK)],
                        rel_sh.at[pl.ds(s * _RCHUNK, _RCHUNK)])

    plsc.subcore_barrier()

    nloc = (_NCHUNKS // _NS) + jnp.where(s < (_NCHUNKS % _NS), 1, 0)

    def make_chunk_body(do_rel):
        def chunk_body(i, carry):
            cid = s + i * _NS
            base = cid * _CHUNK
            idx_src = heads2 if do_rel else tails2
            cps = [
                pltpu.make_async_copy(val2.at[pl.ds(base, _CHUNK)], val_v, sem),
                pltpu.make_async_copy(w1.at[pl.ds(base, _CHUNK)], w_v, sem),
                pltpu.make_async_copy(idx_src.at[pl.ds(cid * _NSUB, _NSUB)],
                                      sidx_v, sem),
            ]
            if do_rel:
                cps.append(pltpu.make_async_copy(
                    rels1.at[pl.ds(base, _CHUNK)], rels_v, sem))
                cps.append(pltpu.make_async_copy(
                    ids1.at[pl.ds(base, _CHUNK)], ids_v, sem))
            for cp in cps:
                cp.start()
            for cp in cps:
                cp.wait()

            def groupfn(g, rcarry):
                gb = g * 16
                w16 = w_v[pl.ds(gb, 16)]
                if do_rel:
                    j = g // 8
                    col = (g % 8) * 16
                    ridx_v[j, pl.ds(col, 16)] = (
                        rels_v[pl.ds(gb, 16)] + ids_v[pl.ds(gb, 16)] * _NREL)
                for r in range(16):
                    val_v[gb + r] = val_v[gb + r] * w16[r]
                return rcarry

            lax.fori_loop(0, _CHUNK // 16, groupfn, 0)

            scs = [pltpu.make_async_copy(val_v.at[pl.ds(j * _SUB, _SUB)],
                                         acc_sh.at[sidx_v.at[j]], sem)
                   for j in range(_NSUB)]
            if do_rel:
                scs += [pltpu.make_async_copy(val_v.at[pl.ds(j * _SUB, _SUB)],
                                              rel_sh.at[ridx_v.at[j]], sem)
                        for j in range(_NSUB)]
            for sc_ in scs:
                sc_.start(add=True)
            for sc_ in scs:
                sc_.wait()
            return carry

        return chunk_body

    @pl.when(c == 0)
    def _():
        lax.fori_loop(0, nloc, make_chunk_body(True), 0)

    @pl.when(c != 0)
    def _():
        lax.fori_loop(0, nloc, make_chunk_body(False), 0)

    plsc.subcore_barrier()

    def make_wb(dst):
        def wb(i, carry):
            b = (s + i * _NS) * _ECHUNK
            pltpu.sync_copy(acc_sh.at[pl.ds(b, _ECHUNK)],
                            dst.at[pl.ds(b, _ECHUNK)])
            return carry

        return wb

    @pl.when(c == 0)
    def _():
        lax.fori_loop(0, nz, make_wb(head_out), 0)

        @pl.when(s < _NRCHUNK)
        def _():
            pltpu.sync_copy(rel_sh.at[pl.ds(s * _RCHUNK, _RCHUNK)],
                            rel_out.at[pl.ds(s * _RCHUNK, _RCHUNK)])

    @pl.when(c != 0)
    def _():
        lax.fori_loop(0, nz, make_wb(tail_out), 0)


def _p2_body(heads2, w1, head_in, tail_in, rel_in, out,
             val_v, w_v, gidx_v, sem):
    c = lax.axis_index("c")
    s = lax.axis_index("s")
    w = s * _NC + c

    ncp = (_NECHUNK // _NW) + jnp.where(w < _NECHUNK % _NW, 1, 0)

    def cbody(i, carry):
        b = (w + i * _NW) * _ECHUNK
        pltpu.sync_copy(head_in.at[pl.ds(b, _ECHUNK)],
                        val_v.at[pl.ds(0, _ECHUNK)])
        pltpu.sync_copy(val_v.at[pl.ds(0, _ECHUNK)],
                        out.at[pl.ds(b, _ECHUNK)])
        pltpu.sync_copy(tail_in.at[pl.ds(b, _ECHUNK)],
                        val_v.at[pl.ds(0, _ECHUNK)])
        pltpu.sync_copy(val_v.at[pl.ds(0, _ECHUNK)],
                        out.at[pl.ds(_NE + b, _ECHUNK)])
        return carry

    lax.fori_loop(0, ncp, cbody, 0)

    @pl.when(w < _NRCHUNK)
    def _():
        pltpu.sync_copy(rel_in.at[pl.ds(w * _RCHUNK, _RCHUNK)],
                        val_v.at[pl.ds(0, _RCHUNK)])
        pltpu.sync_copy(val_v.at[pl.ds(0, _RCHUNK)],
                        out.at[pl.ds(2 * _NE + w * _RCHUNK, _RCHUNK)])

    nloc = (_NCHUNKS // _NW) + jnp.where(w < (_NCHUNKS % _NW), 1, 0)

    def chunk_body(i, carry):
        cid = w + i * _NW
        base = cid * _CHUNK
        cps = [
            pltpu.make_async_copy(heads2.at[pl.ds(cid * _NSUB, _NSUB)],
                                  gidx_v, sem),
            pltpu.make_async_copy(w1.at[pl.ds(base, _CHUNK)], w_v, sem),
        ]
        for cp in cps:
            cp.start()
        for cp in cps:
            cp.wait()

        gs = [pltpu.make_async_copy(head_in.at[gidx_v.at[j]],
                                    val_v.at[pl.ds(j * _SUB, _SUB)], sem)
              for j in range(_NSUB)]
        for g in gs:
            g.start()
        for g in gs:
            g.wait()

        def groupfn(g, rcarry):
            gb = g * 16
            w16 = w_v[pl.ds(gb, 16)]
            for r in range(16):
                val_v[gb + r] = val_v[gb + r] * w16[r]
            return rcarry

        lax.fori_loop(0, _CHUNK // 16, groupfn, 0)
        pltpu.sync_copy(val_v, out.at[pl.ds(_FACT_OFF + base, _CHUNK)])
        return carry

    lax.fori_loop(0, nloc, chunk_body, 0)


def kernel(batch_heads, batch_rels, batch_tails, batch_ids, fact_ids,
           weight_list, fact_val):
    del fact_ids
    heads2 = batch_heads.reshape(_IDXROWS, _SUB)
    tails2 = batch_tails.reshape(_IDXROWS, _SUB)
    mesh = plsc.VectorSubcoreMesh(core_axis_name="c", subcore_axis_name="s")

    f32 = jnp.float32
    cparams = pltpu.CompilerParams(use_tc_tiling_on_sc=False)
    p1 = pl.kernel(
        _p1_body,
        out_type=(
            jax.ShapeDtypeStruct((_NE, _D), f32),
            jax.ShapeDtypeStruct((_NE, _D), f32),
            jax.ShapeDtypeStruct((_NRB, _D), f32),
        ),
        mesh=mesh,
        scratch_types=[
            pltpu.VMEM_SHARED((_NE, _D), f32),
            pltpu.VMEM_SHARED((_NRB, _D), f32),
            pltpu.VMEM((_CHUNK, _D), f32),
            pltpu.VMEM((_CHUNK,), f32),
            pltpu.VMEM((_NSUB, _SUB), jnp.int32),
            pltpu.VMEM((_CHUNK,), jnp.int32),
            pltpu.VMEM((_CHUNK,), jnp.int32),
            pltpu.VMEM((_NSUB, _SUB), jnp.int32),
            pltpu.SemaphoreType.DMA,
        ],
        compiler_params=cparams,
    )
    head_agg, tail_agg, rel_agg = p1(heads2, tails2, batch_rels, batch_ids,
                                     weight_list, fact_val)

    p2 = pl.kernel(
        _p2_body,
        out_type=jax.ShapeDtypeStruct((_OUT_ROWS, _D), f32),
        mesh=mesh,
        scratch_types=[
            pltpu.VMEM((_CHUNK, _D), f32),
            pltpu.VMEM((_CHUNK,), f32),
            pltpu.VMEM((_NSUB, _SUB), jnp.int32),
            pltpu.SemaphoreType.DMA,
        ],
        compiler_params=cparams,
    )
    return p2(heads2, weight_list, head_agg, tail_agg, rel_agg)
